# Initial kernel scaffold; baseline (speedup 1.0000x reference)
import jax
import jax.numpy as jnp
from jax.experimental import pallas as pl


def kernel(pos, pos_embedding):
    raise NotImplementedError("write your pallas kernel here")


# speedup vs baseline: 1.7722x; 1.7722x over previous
import jax
import jax.numpy as jnp
from jax import lax
from jax.experimental import pallas as pl
from jax.experimental.pallas import tpu as pltpu
from jax.experimental.pallas import tpu_sc as plsc

T = 4096
D = 2048
B = 4 * 4096
NC, NS = 2, 16
NW = NC * NS
BPW = B // NW
C = 32
NCH = BPW // C


def _gather_body(idx_hbm, table_hbm, out_hbm, idx_v, buf, gsem):
    wid = lax.axis_index("s") * NC + lax.axis_index("c")
    pltpu.sync_copy(idx_hbm.at[wid], idx_v)

    def step(g, carry):
        pltpu.async_copy(table_hbm.at[idx_v.at[g]], buf, gsem).wait()
        pltpu.sync_copy(buf, out_hbm.at[pl.ds(wid * BPW + g * C, C)])
        return carry

    lax.fori_loop(0, NCH, step, 0)


def kernel(pos, pos_embedding):
    idx = pos.reshape(NW, NCH, C).astype(jnp.int32)
    mesh = plsc.VectorSubcoreMesh(core_axis_name="c", subcore_axis_name="s")
    out = pl.kernel(
        _gather_body,
        mesh=mesh,
        out_type=jax.ShapeDtypeStruct((B, D), jnp.float32),
        scratch_types=[
            pltpu.VMEM((NCH, C), jnp.int32),
            pltpu.VMEM((C, D), jnp.float32),
            pltpu.SemaphoreType.DMA,
        ],
    )(idx, pos_embedding)
    return out.reshape(pos.shape[0], pos.shape[1], D)


# speedup vs baseline: 1.9260x; 1.0868x over previous
import jax
import jax.numpy as jnp
from jax import lax
from jax.experimental import pallas as pl
from jax.experimental.pallas import tpu as pltpu
from jax.experimental.pallas import tpu_sc as plsc

T = 4096
D = 2048
B = 4 * 4096
NC, NS = 2, 16
NW = NC * NS
BPW = B // NW
C = 16
NCH = BPW // C


def _gather_body(idx_hbm, table_hbm, out_hbm, idx_v, buf0, buf1,
                 gsem0, gsem1, ssem0, ssem1):
    wid = lax.axis_index("s") * NC + lax.axis_index("c")
    pltpu.sync_copy(idx_hbm.at[wid], idx_v)
    base = wid * BPW

    bufs = (buf0, buf1)
    gsems = (gsem0, gsem1)
    ssems = (ssem0, ssem1)

    pltpu.async_copy(table_hbm.at[idx_v.at[0]], buf0, gsem0)
    pltpu.async_copy(table_hbm.at[idx_v.at[1]], buf1, gsem1)

    def chain(b, buf, gsem, ssem):
        def step(k, carry):
            g = 2 * k + b
            pltpu.make_async_copy(table_hbm.at[idx_v.at[g]], buf, gsem).wait()
            scat = pltpu.make_async_copy(
                buf, out_hbm.at[pl.ds(base + g * C, C)], ssem)
            scat.start()
            scat.wait()

            @pl.when(g + 2 < NCH)
            def _():
                pltpu.async_copy(table_hbm.at[idx_v.at[g + 2]], buf, gsem)

            return carry
        return step

    def step_pair(k, carry):
        carry = chain(0, buf0, gsem0, ssem0)(k, carry)
        carry = chain(1, buf1, gsem1, ssem1)(k, carry)
        return carry

    lax.fori_loop(0, NCH // 2, step_pair, 0)


def kernel(pos, pos_embedding):
    idx = pos.reshape(NW, NCH, C).astype(jnp.int32)
    mesh = plsc.VectorSubcoreMesh(core_axis_name="c", subcore_axis_name="s")
    out = pl.kernel(
        _gather_body,
        mesh=mesh,
        out_type=jax.ShapeDtypeStruct((B, D), jnp.float32),
        scratch_types=[
            pltpu.VMEM((NCH, C), jnp.int32),
            pltpu.VMEM((C, D), jnp.float32),
            pltpu.VMEM((C, D), jnp.float32),
            pltpu.SemaphoreType.DMA,
            pltpu.SemaphoreType.DMA,
            pltpu.SemaphoreType.DMA,
            pltpu.SemaphoreType.DMA,
        ],
    )(idx, pos_embedding)
    return out.reshape(pos.shape[0], pos.shape[1], D)
